# Initial kernel scaffold; baseline (speedup 1.0000x reference)
import jax
import jax.numpy as jnp
from jax.experimental import pallas as pl


def kernel(points, features, kernel_points, W, neighbor_indices):
    raise NotImplementedError("write your pallas kernel here")


# speedup vs baseline: 1.6091x; 1.6091x over previous
import functools

import jax
import jax.numpy as jnp
from jax import lax
from jax.experimental import pallas as pl
from jax.experimental.pallas import tpu as pltpu
from jax.experimental.pallas import tpu_sc as plsc

N = 10000
DEG = 32
D_IN = 128
D_OUT = 128
KP = 15
RADIUS = 1.0
SIGMA = 0.3 * RADIUS
INV2SIG2 = -1.0 / (2.0 * SIGMA * SIGMA + 1e-9)

NC = 2
NS = 16
NW = NC * NS
NPAD = 10240
RPW = NPAD // NW
C = 4
NCHUNK = RPW // C
KPG = 5
F16 = D_IN // 16


def _sc_weighted_gather(points_flat, idx_flat, features, kp_flat):
    mesh = plsc.VectorSubcoreMesh(core_axis_name="c", subcore_axis_name="s")

    @functools.partial(
        pl.kernel,
        mesh=mesh,
        compiler_params=pltpu.CompilerParams(needs_layout_passes=False),
        out_type=jax.ShapeDtypeStruct((NPAD, KP * D_IN), jnp.float32),
        scratch_types=[
            pltpu.VMEM((NPAD * 4 + 16,), jnp.float32),
            pltpu.VMEM((RPW * DEG,), jnp.int32),
            pltpu.VMEM((C * DEG, D_IN), jnp.float32),
            pltpu.VMEM((C * KP * DEG + 16,), jnp.float32),
            pltpu.VMEM((C, KP * D_IN), jnp.float32),
            pltpu.VMEM((48,), jnp.float32),
            pltpu.SemaphoreType.DMA,
        ],
    )
    def sc_kernel(pts_hbm, idx_hbm, feat_hbm, kp_hbm, wf_hbm,
                  pts_v, idx_v, gbuf, wbuf, wfst, kp_v, sem):
        wid = lax.axis_index("s") * NC + lax.axis_index("c")
        base = wid * RPW

        pltpu.sync_copy(pts_hbm, pts_v)
        pltpu.sync_copy(idx_hbm.at[pl.ds(base * DEG, RPW * DEG)], idx_v)
        pltpu.sync_copy(kp_hbm, kp_v)

        kpvec = [kp_v[pl.ds(16 * i, 16)] for i in range(3)]
        kpc = [kpvec[(p * 3 + c) // 16][(p * 3 + c) % 16]
               for p in range(KP) for c in range(3)]

        def chunk_body(ch, carry):
            idx_sl = idx_v.at[pl.ds(ch * (C * DEG), C * DEG)]
            pltpu.async_copy(feat_hbm.at[idx_sl], gbuf, sem).wait()

            for r in range(C):
                n = base + ch * C + r
                pvec = pts_v[pl.ds(n * 4, 16)]
                px = pvec[0]
                py = pvec[1]
                pz = pvec[2]
                for h in range(2):
                    iv = idx_v[pl.ds((ch * C + r) * DEG + h * 16, 16)]
                    iv4 = iv * 4
                    nx = plsc.load_gather(pts_v, [iv4])
                    ny = plsc.load_gather(pts_v, [iv4 + 1])
                    nz = plsc.load_gather(pts_v, [iv4 + 2])
                    dx0 = nx - px
                    dy0 = ny - py
                    dz0 = nz - pz
                    for p in range(KP):
                        dx = dx0 - kpc[p * 3]
                        dy = dy0 - kpc[p * 3 + 1]
                        dz = dz0 - kpc[p * 3 + 2]
                        sq = dx * dx + dy * dy + dz * dz
                        wbuf[pl.ds((r * KP + p) * DEG + h * 16, 16)] = (
                            jnp.exp(sq * INV2SIG2))

                fbase = r * DEG
                for pg in range(KP // KPG):
                    def kbody(k, acc, pg=pg):
                        f = [gbuf[fbase + k, pl.ds(j * 16, 16)]
                             for j in range(F16)]
                        out = []
                        for pi in range(KPG):
                            p = pg * KPG + pi
                            w = wbuf[pl.ds((r * KP + p) * DEG + k, 16)][0]
                            out.append([acc[pi][j] + w * f[j]
                                        for j in range(F16)])
                        return out
                    acc0 = [[jnp.zeros((16,), jnp.float32)
                             for _ in range(F16)] for _ in range(KPG)]
                    acc = lax.fori_loop(0, DEG, kbody, acc0)
                    for pi in range(KPG):
                        p = pg * KPG + pi
                        for j in range(F16):
                            wfst[r, pl.ds(p * D_IN + j * 16, 16)] = acc[pi][j]

            pltpu.sync_copy(wfst, wf_hbm.at[pl.ds(base + ch * C, C)])
            return carry

        lax.fori_loop(0, NCHUNK, chunk_body, 0)

    return sc_kernel(points_flat, idx_flat, features, kp_flat)


def _tc_matmul(wf, w2):
    blk = 512

    def body(x_ref, w_ref, o_ref):
        o_ref[...] = jnp.dot(x_ref[...], w_ref[...],
                             preferred_element_type=jnp.float32)

    return pl.pallas_call(
        body,
        grid=(NPAD // blk,),
        in_specs=[
            pl.BlockSpec((blk, KP * D_IN), lambda i: (i, 0)),
            pl.BlockSpec((KP * D_IN, D_OUT), lambda i: (0, 0)),
        ],
        out_specs=pl.BlockSpec((blk, D_OUT), lambda i: (i, 0)),
        out_shape=jax.ShapeDtypeStruct((NPAD, D_OUT), jnp.float32),
    )(wf, w2)


def kernel(points, features, kernel_points, W, neighbor_indices):
    points = points.astype(jnp.float32)
    features = features.astype(jnp.float32)
    idx = neighbor_indices.astype(jnp.int32)

    pts_pad = jnp.zeros((NPAD * 4 + 16,), jnp.float32).at[
        :NPAD * 4].set(jnp.zeros((NPAD, 4), jnp.float32)
                       .at[:N, :3].set(points).reshape(-1))
    idx_pad = jnp.zeros((NPAD, DEG), jnp.int32).at[:N].set(idx)
    kp_flat = jnp.zeros((48,), jnp.float32).at[:KP * 3].set(
        kernel_points.reshape(-1).astype(jnp.float32))
    w2 = W.astype(jnp.float32).reshape(KP * D_IN, D_OUT)

    wf = _sc_weighted_gather(pts_pad.reshape(-1), idx_pad.reshape(-1),
                             features, kp_flat)
    out = _tc_matmul(wf, w2)
    return out[:N]


# speedup vs baseline: 1.9975x; 1.2414x over previous
import functools

import jax
import jax.numpy as jnp
from jax import lax
from jax.experimental import pallas as pl
from jax.experimental.pallas import tpu as pltpu
from jax.experimental.pallas import tpu_sc as plsc

N = 10000
DEG = 32
D_IN = 128
D_OUT = 128
KP = 15
RADIUS = 1.0
SIGMA = 0.3 * RADIUS
INV2SIG2 = -1.0 / (2.0 * SIGMA * SIGMA + 1e-9)

NC = 2
NS = 16
NW = NC * NS
NPAD = 10240
RPW = NPAD // NW
C = 4
NCHUNK = RPW // C
KPG = 5
F16 = D_IN // 16


def _sc_weighted_gather(points_flat, idx_flat, features, kp_flat):
    mesh = plsc.VectorSubcoreMesh(core_axis_name="c", subcore_axis_name="s")

    @functools.partial(
        pl.kernel,
        mesh=mesh,
        compiler_params=pltpu.CompilerParams(needs_layout_passes=False),
        out_type=jax.ShapeDtypeStruct((NPAD, KP * D_IN), jnp.float32),
        scratch_types=[
            pltpu.VMEM((NPAD * 4 + 16,), jnp.float32),
            pltpu.VMEM((RPW * DEG,), jnp.int32),
            pltpu.VMEM((2, C * DEG, D_IN), jnp.float32),
            pltpu.VMEM((C * DEG * 16 + 16,), jnp.float32),
            pltpu.VMEM((C, KP * D_IN), jnp.float32),
            pltpu.VMEM((48,), jnp.float32),
            pltpu.SemaphoreType.DMA,
        ],
    )
    def sc_kernel(pts_hbm, idx_hbm, feat_hbm, kp_hbm, wf_hbm,
                  pts_v, idx_v, gbuf, wbuf, wfst, kp_v, sem):
        wid = lax.axis_index("s") * NC + lax.axis_index("c")
        base = wid * RPW

        pltpu.sync_copy(pts_hbm, pts_v)
        pltpu.sync_copy(idx_hbm.at[pl.ds(base * DEG, RPW * DEG)], idx_v)
        pltpu.sync_copy(kp_hbm, kp_v)

        kpvec = [kp_v[pl.ds(16 * i, 16)] for i in range(3)]
        kpc = [kpvec[(p * 3 + c) // 16][(p * 3 + c) % 16]
               for p in range(KP) for c in range(3)]
        kax = [-2.0 * INV2SIG2 * kpc[p * 3] for p in range(KP)]
        kay = [-2.0 * INV2SIG2 * kpc[p * 3 + 1] for p in range(KP)]
        kaz = [-2.0 * INV2SIG2 * kpc[p * 3 + 2] for p in range(KP)]
        kb = [INV2SIG2 * (kpc[p * 3] * kpc[p * 3]
                          + kpc[p * 3 + 1] * kpc[p * 3 + 1]
                          + kpc[p * 3 + 2] * kpc[p * 3 + 2])
              for p in range(KP)]

        lane_ids = lax.iota(jnp.int32, 16)

        def gather_desc(ch, par):
            idx_sl = idx_v.at[pl.ds(ch * (C * DEG), C * DEG)]
            return pltpu.make_async_copy(feat_hbm.at[idx_sl], gbuf.at[par], sem)

        gather_desc(0, 0).start()

        def chunk_body(ch, carry):
            par = lax.rem(ch, 2)
            gather_desc(ch, par).wait()

            @pl.when(ch + 1 < NCHUNK)
            def _():
                gather_desc(ch + 1, 1 - par).start()

            for r in range(C):
                n = base + ch * C + r
                pvec = pts_v[pl.ds(n * 4, 16)]
                px = pvec[0]
                py = pvec[1]
                pz = pvec[2]
                for h in range(2):
                    iv = idx_v[pl.ds((ch * C + r) * DEG + h * 16, 16)]
                    iv4 = iv * 4
                    dx = plsc.load_gather(pts_v, [iv4]) - px
                    dy = plsc.load_gather(pts_v, [iv4 + 1]) - py
                    dz = plsc.load_gather(pts_v, [iv4 + 2]) - pz
                    dd = (dx * dx + dy * dy + dz * dz) * INV2SIG2
                    widx = (r * DEG + h * 16 + lane_ids) * 16
                    for p in range(KP):
                        arg = dd + (kax[p] * dx + kay[p] * dy
                                    + kaz[p] * dz + kb[p])
                        plsc.store_scatter(wbuf, [widx + p], jnp.exp(arg))

                fbase = r * DEG
                for pg in range(KP // KPG):
                    def kbody(k, acc, pg=pg):
                        f = [gbuf[par, fbase + k, pl.ds(j * 16, 16)]
                             for j in range(F16)]
                        wv = wbuf[pl.ds((fbase + k) * 16, 16)]
                        out = []
                        for pi in range(KPG):
                            w = wv[pg * KPG + pi]
                            out.append([acc[pi][j] + w * f[j]
                                        for j in range(F16)])
                        return out
                    acc0 = [[jnp.zeros((16,), jnp.float32)
                             for _ in range(F16)] for _ in range(KPG)]
                    acc = lax.fori_loop(0, DEG, kbody, acc0, unroll=2)
                    for pi in range(KPG):
                        p = pg * KPG + pi
                        for j in range(F16):
                            wfst[r, pl.ds(p * D_IN + j * 16, 16)] = acc[pi][j]

            pltpu.sync_copy(wfst, wf_hbm.at[pl.ds(base + ch * C, C)])
            return carry

        lax.fori_loop(0, NCHUNK, chunk_body, 0)

    return sc_kernel(points_flat, idx_flat, features, kp_flat)


def _tc_matmul(wf, w2):
    blk = 512

    def body(x_ref, w_ref, o_ref):
        o_ref[...] = jnp.dot(x_ref[...], w_ref[...],
                             preferred_element_type=jnp.float32)

    return pl.pallas_call(
        body,
        grid=(NPAD // blk,),
        in_specs=[
            pl.BlockSpec((blk, KP * D_IN), lambda i: (i, 0)),
            pl.BlockSpec((KP * D_IN, D_OUT), lambda i: (0, 0)),
        ],
        out_specs=pl.BlockSpec((blk, D_OUT), lambda i: (i, 0)),
        out_shape=jax.ShapeDtypeStruct((NPAD, D_OUT), jnp.float32),
    )(wf, w2)


def kernel(points, features, kernel_points, W, neighbor_indices):
    points = points.astype(jnp.float32)
    features = features.astype(jnp.float32)
    idx = neighbor_indices.astype(jnp.int32)

    pts_pad = jnp.zeros((NPAD * 4 + 16,), jnp.float32).at[
        :NPAD * 4].set(jnp.zeros((NPAD, 4), jnp.float32)
                       .at[:N, :3].set(points).reshape(-1))
    idx_pad = jnp.zeros((NPAD, DEG), jnp.int32).at[:N].set(idx)
    kp_flat = jnp.zeros((48,), jnp.float32).at[:KP * 3].set(
        kernel_points.reshape(-1).astype(jnp.float32))
    w2 = W.astype(jnp.float32).reshape(KP * D_IN, D_OUT)

    wf = _sc_weighted_gather(pts_pad.reshape(-1), idx_pad.reshape(-1),
                             features, kp_flat)
    out = _tc_matmul(wf, w2)
    return out[:N]
